# baseline (speedup 1.0000x reference)
import functools

import jax
import jax.numpy as jnp
from jax import lax
from jax.experimental import pallas as pl
from jax.experimental.pallas import tpu as pltpu
from jax.experimental.pallas import tpu_sc as plsc

SKIP = 2
DELAY = 4
DIM = 8
LANES = 16
NC = 2
NS = 16
NW = NC * NS


def _build_kernel(n):
  num_pts = (n - (DIM - 1) * DELAY) // SKIP
  c_rows = -(-num_pts // NW)
  c_rows += c_rows % 2
  r_rows = 2048
  nsub = -(-c_rows // r_rows)
  in_len = 2 * r_rows + 40
  nvec = r_rows * DIM // LANES
  out_words = r_rows * DIM

  mesh = plsc.VectorSubcoreMesh(core_axis_name="c", subcore_axis_name="s")

  @functools.partial(
      pl.kernel,
      mesh=mesh,
      out_type=jax.ShapeDtypeStruct((num_pts, DIM), jnp.float32),
      scratch_types=[
          pltpu.VMEM((in_len,), jnp.float32),
          pltpu.VMEM((in_len,), jnp.float32),
          pltpu.VMEM((r_rows, DIM), jnp.float32),
          pltpu.VMEM((r_rows, DIM), jnp.float32),
          pltpu.SemaphoreType.DMA,
          pltpu.SemaphoreType.DMA,
          pltpu.SemaphoreType.DMA,
          pltpu.SemaphoreType.DMA,
      ],
      compiler_params=pltpu.CompilerParams(
          needs_layout_passes=False, use_tc_tiling_on_sc=False),
  )
  def tde(ts_hbm, out_hbm, in0, in1, ob0, ob1, si0, si1, so0, so1):
    ins, obs = [in0, in1], [ob0, ob1]
    sis, sos = [si0, si1], [so0, so1]

    w = lax.axis_index("s") * NC + lax.axis_index("c")
    row0 = jnp.minimum(w * c_rows, num_pts - c_rows)

    def row_at(t):
      return jnp.minimum(row0 + t * r_rows, row0 + c_rows - r_rows)

    def in_start(t):
      row = row_at(t)
      a = jnp.minimum((2 * row) & -8, n - in_len)
      return a, pltpu.async_copy(
          ts_hbm.at[pl.ds(pl.multiple_of(a, 8), in_len)],
          ins[t % 2], sis[t % 2])

    lane = lax.iota(jnp.int32, LANES)
    pat = 2 * (lane >> 3) + 4 * (lane & 7)
    row_pat = lane >> 3
    col_pat = lane & 7

    aligns = [None] * nsub
    in_cp = [None] * nsub
    out_cp = [None] * nsub
    aligns[0], in_cp[0] = in_start(0)
    for t in range(nsub):
      if t + 1 < nsub:
        aligns[t + 1], in_cp[t + 1] = in_start(t + 1)
      in_cp[t].wait()
      row = row_at(t)
      base = (2 * row - aligns[t]) + pat
      src, dst = ins[t % 2], obs[t % 2]
      if t >= 2:
        out_cp[t - 2].wait()

      def body(v, _, base=base, src=src, dst=dst):
        g = plsc.load_gather(src, [base + 4 * v])
        plsc.store_scatter(dst, [row_pat + 2 * v, col_pat], g)
        return 0

      lax.fori_loop(0, nvec, body, 0, unroll=8)
      out_cp[t] = pltpu.async_copy(
          dst, out_hbm.at[pl.ds(row, r_rows)], sos[t % 2])
    out_cp[nsub - 2].wait()
    out_cp[nsub - 1].wait()

  return tde, num_pts


def kernel(timeSeries):
  n = timeSeries.shape[0]
  if n == 1:
    return timeSeries
  tde, num_pts = _build_kernel(n)
  return tde(timeSeries)


# speedup vs baseline: 4.5130x; 4.5130x over previous
import functools

import jax
import jax.numpy as jnp
from jax import lax
from jax.experimental import pallas as pl
from jax.experimental.pallas import tpu as pltpu
from jax.experimental.pallas import tpu_sc as plsc

SKIP = 2
DELAY = 4
DIM = 8
LANES = 16
NC = 2
NS = 16
NW = NC * NS


def _build_kernel(n):
  num_pts = (n - (DIM - 1) * DELAY) // SKIP
  nt = -(-num_pts // 128)
  tpw = -(-nt // NW)
  sb = 16
  nsub = -(-tpw // sb)
  cols = sb * 128
  in_len = 2 * cols + 64
  assert (n - in_len) % 8 == 0

  mesh = plsc.VectorSubcoreMesh(core_axis_name="c", subcore_axis_name="s")

  @functools.partial(
      pl.kernel,
      mesh=mesh,
      out_type=jax.ShapeDtypeStruct((nt, DIM, 128), jnp.float32),
      scratch_types=[
          pltpu.VMEM((in_len,), jnp.float32),
          pltpu.VMEM((in_len,), jnp.float32),
          pltpu.VMEM((sb, DIM, 128), jnp.float32),
          pltpu.VMEM((sb, DIM, 128), jnp.float32),
          pltpu.SemaphoreType.DMA,
          pltpu.SemaphoreType.DMA,
          pltpu.SemaphoreType.DMA,
          pltpu.SemaphoreType.DMA,
      ],
      compiler_params=pltpu.CompilerParams(
          needs_layout_passes=False, use_tc_tiling_on_sc=False),
  )
  def tde(ts_hbm, out_hbm, in0, in1, ob0, ob1, si0, si1, so0, so1):
    ins, obs = [in0, in1], [ob0, ob1]
    sis, sos = [si0, si1], [so0, so1]

    w = lax.axis_index("s") * NC + lax.axis_index("c")
    t0w = jnp.minimum(w * tpw, nt - tpw)

    def tile_at(t):
      return jnp.minimum(t0w + t * sb, t0w + tpw - sb)

    def in_start(t):
      j0 = tile_at(t) * 128
      a = jnp.minimum((SKIP * j0) & -8, n - in_len)
      return a, pltpu.async_copy(
          ts_hbm.at[pl.ds(pl.multiple_of(a, 8), in_len)],
          ins[t % 2], sis[t % 2])

    lane = lax.iota(jnp.int32, LANES)
    pat = SKIP * lane

    aligns = [None] * nsub
    in_cp = [None] * nsub
    out_cp = [None] * nsub
    aligns[0], in_cp[0] = in_start(0)
    for t in range(nsub):
      if t + 1 < nsub:
        aligns[t + 1], in_cp[t + 1] = in_start(t + 1)
      in_cp[t].wait()
      tile0 = tile_at(t)
      off = SKIP * tile0 * 128 - aligns[t]
      src, dst = ins[t % 2], obs[t % 2]
      if t >= 2:
        out_cp[t - 2].wait()

      for k in range(DIM):
        base = (off + DELAY * k) + pat

        def body(v, _, base=base, src=src, dst=dst, k=k):
          idx = jnp.minimum(base + (SKIP * LANES) * v, in_len - 1)
          g = plsc.load_gather(src, [idx])
          dst[v >> 3, k, pl.ds(pl.multiple_of((v & 7) * LANES, LANES),
                               LANES)] = g
          return 0

        lax.fori_loop(0, cols // LANES, body, 0, unroll=8)

      out_cp[t] = pltpu.async_copy(
          dst, out_hbm.at[pl.ds(tile0, sb)], sos[t % 2])
    out_cp[nsub - 2].wait()
    out_cp[nsub - 1].wait()

  return tde, num_pts, nt


def kernel(timeSeries):
  n = timeSeries.shape[0]
  if n == 1:
    return timeSeries
  tde, num_pts, nt = _build_kernel(n)
  p = tde(timeSeries)
  return p.transpose((0, 2, 1)).reshape(nt * 128, DIM)[:num_pts]


# speedup vs baseline: 9.1606x; 2.0298x over previous
import functools

import jax
import jax.numpy as jnp
from jax import lax
from jax.experimental import pallas as pl
from jax.experimental.pallas import tpu as pltpu
from jax.experimental.pallas import tpu_sc as plsc

SKIP = 2
DELAY = 4
DIM = 8
LANES = 16
NC = 2
NS = 16
NW = NC * NS


def _build_kernel(n):
  num_pts = (n - (DIM - 1) * DELAY) // SKIP
  nt = -(-num_pts // 128)
  tpw = -(-nt // NW)
  sb = 16
  nsub = -(-tpw // sb)
  cols = sb * 128
  in_len = 2 * cols + 64
  assert (n - in_len) % 8 == 0

  mesh = plsc.VectorSubcoreMesh(core_axis_name="c", subcore_axis_name="s")

  @functools.partial(
      pl.kernel,
      mesh=mesh,
      out_type=jax.ShapeDtypeStruct((nt, DIM, 128), jnp.float32),
      scratch_types=[
          pltpu.VMEM((in_len,), jnp.float32),
          pltpu.VMEM((in_len,), jnp.float32),
          pltpu.VMEM((sb, DIM, 128), jnp.float32),
          pltpu.VMEM((sb, DIM, 128), jnp.float32),
          pltpu.SemaphoreType.DMA,
          pltpu.SemaphoreType.DMA,
          pltpu.SemaphoreType.DMA,
          pltpu.SemaphoreType.DMA,
      ],
      compiler_params=pltpu.CompilerParams(
          needs_layout_passes=False, use_tc_tiling_on_sc=False),
  )
  def tde(ts_hbm, out_hbm, in0, in1, ob0, ob1, si0, si1, so0, so1):
    ins, obs = [in0, in1], [ob0, ob1]
    sis, sos = [si0, si1], [so0, so1]

    w = lax.axis_index("s") * NC + lax.axis_index("c")
    t0w = jnp.minimum(w * tpw, nt - tpw)

    def tile_at(t):
      return jnp.minimum(t0w + t * sb, t0w + tpw - sb)

    def in_start(t):
      j0 = tile_at(t) * 128
      a = jnp.minimum((SKIP * j0) & -8, n - in_len)
      return a, pltpu.async_copy(
          ts_hbm.at[pl.ds(pl.multiple_of(a, 8), in_len)],
          ins[t % 2], sis[t % 2])

    lane = lax.iota(jnp.int32, LANES)
    pat = SKIP * lane

    aligns = [None] * nsub
    in_cp = [None] * nsub
    out_cp = [None] * nsub
    aligns[0], in_cp[0] = in_start(0)
    for t in range(nsub):
      if t + 1 < nsub:
        aligns[t + 1], in_cp[t + 1] = in_start(t + 1)
      in_cp[t].wait()
      tile0 = tile_at(t)
      off = SKIP * tile0 * 128 - aligns[t]
      src, dst = ins[t % 2], obs[t % 2]
      if t >= 2:
        out_cp[t - 2].wait()

      @plsc.parallel_loop(0, cols // LANES, carry=off + pat)
      def body(v, idxb, src=src, dst=dst):
        tt = v >> 3
        cb = (v & 7) * LANES
        for k in range(DIM):
          idx = jnp.minimum(idxb + DELAY * k, in_len - 1)
          g = plsc.load_gather(src, [idx])
          dst[tt, k, pl.ds(pl.multiple_of(cb, LANES), LANES)] = g
        return idxb + SKIP * LANES

      out_cp[t] = pltpu.async_copy(
          dst, out_hbm.at[pl.ds(tile0, sb)], sos[t % 2])
    out_cp[nsub - 2].wait()
    out_cp[nsub - 1].wait()

  return tde, num_pts, nt


def kernel(timeSeries):
  n = timeSeries.shape[0]
  if n == 1:
    return timeSeries
  tde, num_pts, nt = _build_kernel(n)
  p = tde(timeSeries)
  return p.transpose((0, 2, 1)).reshape(nt * 128, DIM)[:num_pts]
